# Initial kernel scaffold; baseline (speedup 1.0000x reference)
import functools

import jax
import jax.numpy as jnp
from jax import lax
from jax.experimental import pallas as pl
from jax.experimental.pallas import tpu as pltpu
from jax.experimental.pallas import tpu_sc as plsc

B = 16384
F = 26
D = 32
N = B * F

NC, NS = 2, 16
NW = NC * NS
PER_W = N // NW
CHUNK = 26 * 64
STEPS = PER_W // CHUNK
SUB = 128
NSUB = CHUNK // SUB

_mesh = plsc.VectorSubcoreMesh(core_axis_name="c", subcore_axis_name="s")


@functools.partial(
    pl.kernel,
    out_type=jax.ShapeDtypeStruct((N, D), jnp.float32),
    mesh=_mesh,
    scratch_types=[
        pltpu.VMEM((2, NSUB, SUB), jnp.int32),
        pltpu.VMEM((2, CHUNK, D), jnp.float32),
        pltpu.VMEM((F, D), jnp.float32),
        pltpu.SemaphoreType.DMA,
        pltpu.SemaphoreType.DMA,
        pltpu.SemaphoreType.DMA,
        pltpu.SemaphoreType.DMA,
    ],
)
def _embed(x_hbm, table_hbm, bias_hbm, out_hbm, idx_v, rows_v, bias_v,
           gsem0, gsem1, ssem0, ssem1):
    gsems = (gsem0, gsem1)
    ssems = (ssem0, ssem1)
    wid = lax.axis_index("s") * NC + lax.axis_index("c")

    pltpu.sync_copy(bias_hbm, bias_v)

    def start_chunk(g):
        p = g % 2
        pltpu.sync_copy(x_hbm.at[wid * STEPS + g], idx_v.at[p])
        descs = []
        for j in range(NSUB):
            descs.append(pltpu.async_copy(
                table_hbm.at[idx_v.at[p, j]],
                rows_v.at[p, pl.ds(j * SUB, SUB)],
                gsems[p]))
        return descs

    def add_bias(p):
        def body(i, carry):
            r0 = i * F
            for f in range(F):
                for h in range(2):
                    sl = pl.ds(h * 16, 16)
                    rows_v[p, r0 + f, sl] = rows_v[p, r0 + f, sl] + bias_v[f, sl]
            return carry
        lax.fori_loop(0, CHUNK // F, body, 0)

    gdescs = [None, None]
    sdescs = [None, None]
    gdescs[0] = start_chunk(0)
    for g in range(STEPS):
        p = g % 2
        if g + 1 < STEPS:
            if sdescs[1 - p] is not None:
                sdescs[1 - p].wait()
            gdescs[1 - p] = start_chunk(g + 1)
        for d in gdescs[p]:
            d.wait()
        add_bias(p)
        sdescs[p] = pltpu.async_copy(
            rows_v.at[p],
            out_hbm.at[pl.ds(wid * PER_W + g * CHUNK, CHUNK)],
            ssems[p])
    sdescs[0].wait()
    sdescs[1].wait()


def kernel(x, table, bias):
    xr = x.astype(jnp.int32).reshape(NW * STEPS, NSUB, SUB)
    out = _embed(xr, table, bias)
    return out.reshape(B, F, D)


# speedup vs baseline: 1.3820x; 1.3820x over previous
import functools

import jax
import jax.numpy as jnp
from jax import lax
from jax.experimental import pallas as pl
from jax.experimental.pallas import tpu as pltpu
from jax.experimental.pallas import tpu_sc as plsc

B = 16384
F = 26
D = 32
N = B * F

NC, NS = 2, 16
NW = NC * NS
PER_W = N // NW
CHUNK = 26 * 64
STEPS = PER_W // CHUNK
SUB = 128
NSUB = CHUNK // SUB

_mesh = plsc.VectorSubcoreMesh(core_axis_name="c", subcore_axis_name="s")


@functools.partial(
    pl.kernel,
    out_type=jax.ShapeDtypeStruct((N, D), jnp.float32),
    mesh=_mesh,
    compiler_params=pltpu.CompilerParams(use_tc_tiling_on_sc=False),
    scratch_types=[
        pltpu.VMEM((2, NSUB, SUB), jnp.int32),
        pltpu.VMEM((2, CHUNK, D), jnp.float32),
        pltpu.VMEM((F, D), jnp.float32),
        pltpu.SemaphoreType.DMA,
        pltpu.SemaphoreType.DMA,
        pltpu.SemaphoreType.DMA,
        pltpu.SemaphoreType.DMA,
    ],
)
def _embed(x_hbm, table_hbm, bias_hbm, out_hbm, idx_v, rows_v, bias_v,
           gsem0, gsem1, ssem0, ssem1):
    gsems = (gsem0, gsem1)
    ssems = (ssem0, ssem1)
    wid = lax.axis_index("s") * NC + lax.axis_index("c")

    pltpu.sync_copy(bias_hbm, bias_v)

    def start_chunk(g):
        p = g % 2
        pltpu.sync_copy(x_hbm.at[wid * STEPS + g], idx_v.at[p])
        descs = []
        for j in range(NSUB):
            descs.append(pltpu.async_copy(
                table_hbm.at[idx_v.at[p, j]],
                rows_v.at[p, pl.ds(j * SUB, SUB)],
                gsems[p]))
        return descs

    def add_bias(p):
        def body(i, carry):
            r0 = i * F
            for f in range(F):
                for h in range(2):
                    sl = pl.ds(h * 16, 16)
                    rows_v[p, r0 + f, sl] = rows_v[p, r0 + f, sl] + bias_v[f, sl]
            return carry
        lax.fori_loop(0, CHUNK // F, body, 0)

    gdescs = [None, None]
    sdescs = [None, None]
    gdescs[0] = start_chunk(0)
    for g in range(STEPS):
        p = g % 2
        if g + 1 < STEPS:
            if sdescs[1 - p] is not None:
                sdescs[1 - p].wait()
            gdescs[1 - p] = start_chunk(g + 1)
        for d in gdescs[p]:
            d.wait()
        add_bias(p)
        sdescs[p] = pltpu.async_copy(
            rows_v.at[p],
            out_hbm.at[pl.ds(wid * PER_W + g * CHUNK, CHUNK)],
            ssems[p])
    sdescs[0].wait()
    sdescs[1].wait()


def kernel(x, table, bias):
    xr = x.astype(jnp.int32).reshape(NW * STEPS, NSUB, SUB)
    out = _embed(xr, table, bias)
    return out.reshape(B, F, D)


# speedup vs baseline: 1.4589x; 1.0556x over previous
import functools

import jax
import jax.numpy as jnp
from jax import lax
from jax.experimental import pallas as pl
from jax.experimental.pallas import tpu as pltpu
from jax.experimental.pallas import tpu_sc as plsc

B = 16384
F = 26
D = 32
N = B * F

NC, NS = 2, 16
NW = NC * NS
B_PER_W = B // NW
BB = 32
NBLK = B_PER_W // BB
CHUNK = BB * F
SUB = 104
NSUB = CHUNK // SUB

_mesh = plsc.VectorSubcoreMesh(core_axis_name="c", subcore_axis_name="s")


@functools.partial(
    pl.kernel,
    out_type=jax.ShapeDtypeStruct((F, D // 8, B // 128, 8, 128), jnp.float32),
    mesh=_mesh,
    compiler_params=pltpu.CompilerParams(
        use_tc_tiling_on_sc=False, needs_layout_passes=False),
    scratch_types=[
        pltpu.VMEM((2, NSUB, SUB), jnp.int32),
        pltpu.VMEM((2, CHUNK, D), jnp.float32),
        pltpu.VMEM((2, F, D // 8, 8, BB), jnp.float32),
        pltpu.VMEM((F * D * 16,), jnp.float32),
        pltpu.SemaphoreType.DMA,
        pltpu.SemaphoreType.DMA,
        pltpu.SemaphoreType.DMA,
        pltpu.SemaphoreType.DMA,
    ],
)
def _embed(x_hbm, table_hbm, bias_hbm, out_hbm, idx_v, rows_v, o_v, bias_v,
           gsem0, gsem1, osem0, osem1):
    gsems = (gsem0, gsem1)
    osems = (osem0, osem1)
    wid = lax.axis_index("s") * NC + lax.axis_index("c")

    pltpu.sync_copy(bias_hbm, bias_v)

    def start_block(blk):
        p = blk % 2
        pltpu.sync_copy(x_hbm.at[wid * NBLK + blk], idx_v.at[p])
        descs = []
        for j in range(NSUB):
            descs.append(pltpu.async_copy(
                table_hbm.at[idx_v.at[p, j]],
                rows_v.at[p, pl.ds(j * SUB, SUB)],
                gsems[p]))
        return descs

    def compute_block(p):
        def f_body(f, carry):
            rq0 = lax.iota(jnp.int32, 16) * F + f
            rq1 = rq0 + 16 * F

            def d_body(d, carry2):
                dt = d // 8
                dr = d % 8
                cols = jnp.full((16,), d, jnp.int32)
                bv = bias_v[pl.ds((f * D + d) * 16, 16)]
                v0 = plsc.load_gather(rows_v.at[p], [rq0, cols]) + bv
                v1 = plsc.load_gather(rows_v.at[p], [rq1, cols]) + bv
                o_v[p, f, dt, dr, pl.ds(0, 16)] = v0
                o_v[p, f, dt, dr, pl.ds(16, 16)] = v1
                return carry2
            lax.fori_loop(0, D, d_body, 0)
            return carry
        lax.fori_loop(0, F, f_body, 0)

    gdescs = [None, None]
    odescs = [None, None]
    gdescs[0] = start_block(0)
    for blk in range(NBLK):
        p = blk % 2
        if blk + 1 < NBLK:
            gdescs[1 - p] = start_block(blk + 1)
        for dsc in gdescs[p]:
            dsc.wait()
        if odescs[p] is not None:
            odescs[p].wait()
        compute_block(p)
        bt = wid * 4 + blk // 4
        c0 = (blk % 4) * BB
        odescs[p] = pltpu.async_copy(
            o_v.at[p],
            out_hbm.at[:, :, bt, :, pl.ds(c0, BB)],
            osems[p])
    odescs[0].wait()
    odescs[1].wait()


def kernel(x, table, bias):
    xr = x.astype(jnp.int32).reshape(NW * NBLK, NSUB, SUB)
    bias_splat = jnp.broadcast_to(
        bias.reshape(F * D, 1), (F * D, 16)).reshape(F * D * 16)
    out = _embed(xr, table, bias_splat)
    return out.transpose(2, 4, 0, 1, 3).reshape(B, F, D)
